# Initial kernel scaffold; baseline (speedup 1.0000x reference)
import jax
import jax.numpy as jnp
from jax.experimental import pallas as pl


def kernel(w1, b1, w2, b2, w3, b3, wfc, bfc, wh, bh, x):
    raise NotImplementedError("write your pallas kernel here")


# speedup vs baseline: 1.3535x; 1.3535x over previous
import jax
import jax.numpy as jnp
from jax.experimental import pallas as pl
from jax.experimental.pallas import tpu as pltpu

_STATE = 4
_ACT = 6
_W = 21
_C0, _C1, _C2, _C3 = 64, 32, 64, 64
_M1 = 19 * _W + 20
_M2 = 8 * _W + 9
_B = 4


def _feat_kernel(x_ref, w1_ref, b1_ref, w2_ref, b2_ref, w3_ref, b3_ref,
                 o_ref, y1_ref, buf2_ref, y2_ref, buf3_ref):
    for b in range(_B):
        acc = None
        for p in range(2):
            for q in range(2):
                xs = x_ref[b, pl.ds(p * _W + q, _M1), :]
                c = jnp.dot(xs, w1_ref[p * 2 + q],
                            preferred_element_type=jnp.float32)
                acc = c if acc is None else acc + c
        y1_ref[b] = jnp.maximum(acc + b1_ref[...], 0.0)

        for p in range(4):
            for q in range(4):
                t = p * 4 + q
                buf2_ref[b, :, pl.ds(t * _C1, _C1)] = (
                    y1_ref[b, pl.ds(p * _W + q, _M2, stride=2), :])
        y2_ref[b] = jnp.maximum(
            jnp.dot(buf2_ref[b], w2_ref[...],
                    preferred_element_type=jnp.float32) + b2_ref[...], 0.0)

        for p in range(3):
            for q in range(3):
                t = p * 3 + q
                for oh in range(7):
                    buf3_ref[b, pl.ds(oh * 7, 7), pl.ds(t * _C2, _C2)] = (
                        y2_ref[b, pl.ds((oh + p) * _W + q, 7), :])
        o_ref[b] = jnp.maximum(
            jnp.dot(buf3_ref[b], w3_ref[...],
                    preferred_element_type=jnp.float32) + b3_ref[...], 0.0)


def _head_kernel(x_ref, wfc_ref, bfc_ref, wh_ref, bh_ref, logp_ref, val_ref):
    feat = jnp.dot(x_ref[...], wfc_ref[...],
                   preferred_element_type=jnp.float32)
    feat = jnp.maximum(feat + bfc_ref[...], 0.0)
    heads = jnp.dot(feat, wh_ref[...],
                    preferred_element_type=jnp.float32) + bh_ref[...]
    logits = heads[:, :_ACT]
    m = jnp.max(logits, axis=1, keepdims=True)
    z = logits - m
    lse = jnp.log(jnp.sum(jnp.exp(z), axis=1, keepdims=True))
    logp_ref[...] = z - lse
    val_ref[...] = heads[:, _ACT:]


def kernel(w1, b1, w2, b2, w3, b3, wfc, bfc, wh, bh, x):
    n = x.shape[0]
    x1 = x.reshape(n, _STATE, _W, 4, _W, 4).transpose(0, 2, 4, 1, 3, 5)
    x1 = x1.reshape(n, _W * _W, _C0)
    w2c = w2.reshape(16 * _C1, _C2)
    w3c = w3.reshape(9 * _C2, _C3)

    feat = pl.pallas_call(
        _feat_kernel,
        out_shape=jax.ShapeDtypeStruct((n, 49, _C3), jnp.float32),
        grid=(n // _B,),
        in_specs=[
            pl.BlockSpec((_B, _W * _W, _C0), lambda i: (i, 0, 0)),
            pl.BlockSpec((4, _C0, _C1), lambda i: (0, 0, 0)),
            pl.BlockSpec((1, _C1), lambda i: (0, 0)),
            pl.BlockSpec((16 * _C1, _C2), lambda i: (0, 0)),
            pl.BlockSpec((1, _C2), lambda i: (0, 0)),
            pl.BlockSpec((9 * _C2, _C3), lambda i: (0, 0)),
            pl.BlockSpec((1, _C3), lambda i: (0, 0)),
        ],
        out_specs=pl.BlockSpec((_B, 49, _C3), lambda i: (i, 0, 0)),
        scratch_shapes=[
            pltpu.VMEM((_B, _M1, _C1), jnp.float32),
            pltpu.VMEM((_B, _M2, 16 * _C1), jnp.float32),
            pltpu.VMEM((_B, _M2, _C2), jnp.float32),
            pltpu.VMEM((_B, 49, 9 * _C2), jnp.float32),
        ],
        compiler_params=pltpu.CompilerParams(
            dimension_semantics=("parallel",),
            vmem_limit_bytes=48 * 1024 * 1024),
    )(x1, w1, b1, w2c, b2, w3c, b3)

    flat = feat.reshape(n, 49 * _C3)

    h = wfc.shape[1]
    a1 = wh.shape[1]
    tm = 128
    logp, val = pl.pallas_call(
        _head_kernel,
        out_shape=(jax.ShapeDtypeStruct((n, a1 - 1), jnp.float32),
                   jax.ShapeDtypeStruct((n, 1), jnp.float32)),
        grid=(pl.cdiv(n, tm),),
        in_specs=[
            pl.BlockSpec((tm, 49 * _C3), lambda i: (i, 0)),
            pl.BlockSpec((49 * _C3, h), lambda i: (0, 0)),
            pl.BlockSpec((1, h), lambda i: (0, 0)),
            pl.BlockSpec((h, a1), lambda i: (0, 0)),
            pl.BlockSpec((1, a1), lambda i: (0, 0)),
        ],
        out_specs=(pl.BlockSpec((tm, a1 - 1), lambda i: (i, 0)),
                   pl.BlockSpec((tm, 1), lambda i: (i, 0))),
        compiler_params=pltpu.CompilerParams(
            dimension_semantics=("parallel",),
            vmem_limit_bytes=48 * 1024 * 1024),
    )(flat, wfc, bfc, wh, bh)
    return logp, val
